# baseline (speedup 1.0000x reference)
import functools
import math

import jax
import jax.numpy as jnp
from jax import lax
from jax.experimental import pallas as pl
from jax.experimental.pallas import tpu as pltpu
from jax.experimental.pallas import tpu_sc as plsc

NC = 2
NS = 16
NW = NC * NS
K = 128
G = 64
EPS = 1e-5
BR = 1000


def _sc_mesh():
    return plsc.VectorSubcoreMesh(
        core_axis_name="c", subcore_axis_name="s", num_cores=NC, num_subcores=NS
    )


def _make_deg_kernel(C, NP, rps):
    W = 128
    cks = [(o, min(K, rps - o)) for o in range(0, rps, K)]

    NSEM = 4

    @functools.partial(
        pl.kernel,
        out_type=jax.ShapeDtypeStruct((NC, NP, W), jnp.float32),
        mesh=_sc_mesh(),
        scratch_types=[
            pltpu.VMEM((C, K), jnp.int32),
            pltpu.VMEM((K, W), jnp.float32),
            pltpu.VMEM((K, W), jnp.float32),
            pltpu.MemorySpace.VMEM_SHARED((NP, W), jnp.float32),
            [pltpu.SemaphoreType.DMA] * NSEM,
        ],
    )
    def deg_kernel(dst_hbm, out_hbm, didx, onesb, zb, acc, ssem):
        c = lax.axis_index("c")
        s = lax.axis_index("s")
        w = c * NS + s
        base = s * rps

        pltpu.sync_copy(dst_hbm.at[w], didx)

        def fill(i, _):
            for jj in range(W // 16):
                onesb[i, pl.ds(jj * 16, 16)] = jnp.full((16,), 1.0, jnp.float32)
                zb[i, pl.ds(jj * 16, 16)] = jnp.zeros((16,), jnp.float32)
            return 0

        lax.fori_loop(0, K, fill, 0)
        for o, sz in cks:
            pltpu.sync_copy(zb.at[pl.ds(0, sz)], acc.at[pl.ds(base + o, sz)])
        plsc.subcore_barrier()

        def scat_start(j, b):
            pltpu.async_copy(onesb, acc.at[didx.at[j]], ssem[b], add=True)

        def scat_wait(j, b):
            pltpu.make_async_copy(onesb, acc.at[didx.at[j]], ssem[b]).wait()

        def group(g, _):
            for b in range(NSEM):
                j = g * NSEM + b

                @pl.when(j < C)
                def _():
                    @pl.when(j >= NSEM)
                    def _():
                        scat_wait(j - NSEM, b)

                    scat_start(j, b)

            return 0

        lax.fori_loop(0, -(-C // NSEM), group, 0)
        for i in range(min(NSEM, C)):
            j = C - min(NSEM, C) + i
            scat_wait(j, j % NSEM)
        plsc.subcore_barrier()
        for o, sz in cks:
            pltpu.sync_copy(
                acc.at[pl.ds(base + o, sz)],
                out_hbm.at[c].at[pl.ds(base + o, sz)],
            )

    return deg_kernel


def _make_edge_kernel(C, NP, rps, H, KE):
    cks = [(o, min(KE, rps - o)) for o in range(0, rps, KE)]

    NBUF = 3
    LOOK = 1
    NE = 4
    CE = -(-C // NE)

    @functools.partial(
        pl.kernel,
        out_type=jax.ShapeDtypeStruct((NC, NP, H), jnp.float32),
        mesh=_sc_mesh(),
        scratch_types=[
            pltpu.VMEM((CE * KE,), jnp.int32),
            pltpu.VMEM((CE, KE), jnp.int32),
            pltpu.VMEM((NBUF, KE, H), jnp.float32),
            pltpu.MemorySpace.VMEM_SHARED((NP, H), jnp.float32),
            [pltpu.SemaphoreType.DMA] * NBUF,
            [pltpu.SemaphoreType.DMA] * NBUF,
        ],
    )
    def edge_kernel(zp_hbm, src_hbm, dst_hbm, out_hbm, sidx, didx, rows, acc,
                    gsem, ssem):
        c = lax.axis_index("c")
        s = lax.axis_index("s")
        w = c * NS + s
        base = s * rps

        def zfill(i, _):
            for jj in range(H // 16):
                rows[0, i, pl.ds(jj * 16, 16)] = jnp.zeros((16,), jnp.float32)
            return 0

        lax.fori_loop(0, KE, zfill, 0)
        for o, sz in cks:
            pltpu.sync_copy(rows.at[0].at[pl.ds(0, sz)],
                            acc.at[pl.ds(base + o, sz)])
        plsc.subcore_barrier()

        def gather_start(j, b):
            pltpu.async_copy(zp_hbm.at[sidx.at[pl.ds(j * KE, KE)]],
                             rows.at[b], gsem[b])

        def gather_wait(j, b):
            pltpu.make_async_copy(zp_hbm.at[sidx.at[pl.ds(j * KE, KE)]],
                                  rows.at[b], gsem[b]).wait()

        def scat_start(j, b):
            pltpu.async_copy(rows.at[b], acc.at[didx.at[j]], ssem[b],
                             add=True)

        def scat_wait(j, b):
            pltpu.make_async_copy(rows.at[b], acc.at[didx.at[j]],
                                  ssem[b]).wait()

        for e in range(NE):
            ne = min(CE, C - e * CE)
            if ne <= 0:
                break
            pltpu.sync_copy(
                src_hbm.at[pl.ds((w * C + e * CE) * KE, ne * KE)],
                sidx.at[pl.ds(0, ne * KE)])
            pltpu.sync_copy(dst_hbm.at[w, pl.ds(e * CE, ne)],
                            didx.at[pl.ds(0, ne)])
            for j0 in range(min(LOOK, ne)):
                gather_start(j0, j0 % NBUF)

            def group(g, _):
                for b in range(NBUF):
                    j = g * NBUF + b

                    @pl.when(j < ne)
                    def _():
                        gather_wait(j, b)
                        scat_start(j, b)
                        jn = j + LOOK
                        bn = (b + LOOK) % NBUF

                        @pl.when(jn < ne)
                        def _():
                            @pl.when(jn >= NBUF)
                            def _():
                                scat_wait(jn - NBUF, bn)

                            gather_start(jn, bn)

                return 0

            lax.fori_loop(0, -(-ne // NBUF), group, 0)
            for i in range(min(NBUF, ne)):
                j = ne - min(NBUF, ne) + i
                scat_wait(j, j % NBUF)

        plsc.subcore_barrier()
        for o, sz in cks:
            pltpu.sync_copy(
                acc.at[pl.ds(base + o, sz)],
                out_hbm.at[c].at[pl.ds(base + o, sz)],
            )

    return edge_kernel


def _dinv_block(degp):
    d = degp[0, :, 0:1] + degp[1, :, 0:1] + 1.0
    return lax.rsqrt(d)


def _tc_first(degp, x_ref, w0, zp_out):
    dinv = _dinv_block(degp)
    zp_out[...] = dinv * jnp.dot(
        x_ref[...], w0[...], preferred_element_type=jnp.float32
    )


def _tc_layer(degp, part, zp, b, gbn, beta, wn, zp_out):
    dinv = _dinv_block(degp)
    sagg = part[0] + part[1] + zp[...]
    h = jnp.maximum(dinv * sagg + b[...], 0.0)
    h = h * gbn[...] + beta[...]
    zp_out[...] = dinv * jnp.dot(h, wn[...], preferred_element_type=jnp.float32)


def _tc_final(degp, part, zp, b, gbn, beta, bat, w1, b1, w2, b2, out,
              accp, accc):
    i = pl.program_id(0)
    dinv = _dinv_block(degp)
    sagg = part[0] + part[1] + zp[...]
    h = jnp.maximum(dinv * sagg + b[...], 0.0)
    h = h * gbn[...] + beta[...]

    bt = bat[0, 0, :]
    bt2 = jnp.broadcast_to(bt[None, :], (G, BR))
    gi = lax.broadcasted_iota(jnp.int32, (G, BR), 0)
    onehot = (bt2 == gi).astype(jnp.float32)
    pp = jnp.dot(onehot, h, preferred_element_type=jnp.float32)
    cc = jnp.sum(onehot, axis=1, keepdims=True)

    @pl.when(i == 0)
    def _():
        accp[...] = pp
        accc[...] = jnp.broadcast_to(cc, accc.shape)

    @pl.when(i > 0)
    def _():
        accp[...] = accp[...] + pp
        accc[...] = accc[...] + cc

    @pl.when(i == pl.num_programs(0) - 1)
    def _():
        pooled = accp[...] / jnp.maximum(accc[...], 1.0)
        z1 = jnp.maximum(
            jnp.dot(pooled, w1[...], preferred_element_type=jnp.float32) + b1[...],
            0.0,
        )
        out[...] = (
            jnp.dot(z1, w2[...], preferred_element_type=jnp.float32) + b2[...]
        )


def kernel(x, edge_index, batch, conv_W, conv_b, bn_gamma, bn_beta,
           head_W1, head_b1, head_W2, head_b2):
    N, D = x.shape
    L, _, H = conv_W.shape
    OUT = head_W2.shape[1]
    E = edge_index.shape[1]

    KE = 112
    C = -(-E // (NW * K))
    EP = NW * K * C
    src0, dst0 = edge_index[0], edge_index[1]
    dst_r = jnp.concatenate(
        [dst0, jnp.full((EP - E,), N, jnp.int32)]).reshape(NW, C, K)
    CEdg = -(-E // (NW * KE))
    CEdg = -(-CEdg // 32) * 32
    EPe = NW * KE * CEdg
    src_r = jnp.concatenate(
        [src0, jnp.zeros((EPe - E,), jnp.int32)])
    dst_re = jnp.concatenate(
        [dst0, jnp.full((EPe - E,), N, jnp.int32)]).reshape(NW, CEdg, KE)

    rps = -(-(N + 1) // NS)
    rps = -(-rps // K) * K
    NP = NS * rps

    NB = N // BR

    deg_k = _make_deg_kernel(C, NP, rps)
    edge_k = _make_edge_kernel(CEdg, NP, rps, H, KE)

    degp = deg_k(dst_r)

    inv_s = 1.0 / math.sqrt(1.0 + EPS)
    gbn = (bn_gamma * inv_s).reshape(L, 1, H)
    beta = bn_beta.reshape(L, 1, H)
    bias = conv_b.reshape(L, 1, H)
    bat3 = batch.reshape(NB, 1, BR)

    spec_degp = pl.BlockSpec((NC, BR, 128), lambda i: (0, i, 0))
    spec_part = pl.BlockSpec((NC, BR, H), lambda i: (0, i, 0))
    spec_rows = pl.BlockSpec((BR, H), lambda i: (i, 0))
    spec_w = pl.BlockSpec((H, H), lambda i: (0, 0))
    spec_v = pl.BlockSpec((1, H), lambda i: (0, 0))

    zp = pl.pallas_call(
        _tc_first,
        grid=(NB,),
        in_specs=[spec_degp, spec_rows, spec_w],
        out_specs=spec_rows,
        out_shape=jax.ShapeDtypeStruct((N, H), jnp.float32),
    )(degp, x, conv_W[0])

    for i in range(L - 1):
        part = edge_k(zp, src_r, dst_re)
        zp = pl.pallas_call(
            _tc_layer,
            grid=(NB,),
            in_specs=[spec_degp, spec_part, spec_rows, spec_v, spec_v,
                      spec_v, spec_w],
            out_specs=spec_rows,
            out_shape=jax.ShapeDtypeStruct((N, H), jnp.float32),
        )(degp, part, zp, bias[i], gbn[i], beta[i], conv_W[i + 1])

    part = edge_k(zp, src_r, dst_re)
    out = pl.pallas_call(
        _tc_final,
        grid=(NB,),
        in_specs=[spec_degp, spec_part, spec_rows, spec_v, spec_v, spec_v,
                  pl.BlockSpec((1, 1, BR), lambda i: (i, 0, 0)),
                  spec_w, spec_v,
                  pl.BlockSpec((H, OUT), lambda i: (0, 0)),
                  pl.BlockSpec((1, OUT), lambda i: (0, 0))],
        out_specs=pl.BlockSpec((G, OUT), lambda i: (0, 0)),
        out_shape=jax.ShapeDtypeStruct((G, OUT), jnp.float32),
        scratch_shapes=[pltpu.VMEM((G, H), jnp.float32),
                        pltpu.VMEM((G, H), jnp.float32)],
    )(degp, part, zp, bias[L - 1], gbn[L - 1], beta[L - 1], bat3,
      head_W1, head_b1.reshape(1, H), head_W2, head_b2.reshape(1, OUT))
    return out


# speedup vs baseline: 2.2688x; 2.2688x over previous
import functools
import math

import jax
import jax.numpy as jnp
from jax import lax
from jax.experimental import pallas as pl
from jax.experimental.pallas import tpu as pltpu
from jax.experimental.pallas import tpu_sc as plsc

NC = 2
NS = 16
NW = NC * NS
K = 128
G = 64
EPS = 1e-5
BR = 1000


def _sc_mesh():
    return plsc.VectorSubcoreMesh(
        core_axis_name="c", subcore_axis_name="s", num_cores=NC, num_subcores=NS
    )


def _make_deg_kernel(C, NP, rps):
    W = 128
    cks = [(o, min(K, rps - o)) for o in range(0, rps, K)]

    NSEM = 4

    @functools.partial(
        pl.kernel,
        out_type=jax.ShapeDtypeStruct((NC, NP, W), jnp.float32),
        mesh=_sc_mesh(),
        scratch_types=[
            pltpu.VMEM((C, K), jnp.int32),
            pltpu.VMEM((K, W), jnp.float32),
            pltpu.VMEM((K, W), jnp.float32),
            pltpu.MemorySpace.VMEM_SHARED((NP, W), jnp.float32),
            [pltpu.SemaphoreType.DMA] * NSEM,
        ],
    )
    def deg_kernel(dst_hbm, out_hbm, didx, onesb, zb, acc, ssem):
        c = lax.axis_index("c")
        s = lax.axis_index("s")
        w = c * NS + s
        base = s * rps

        pltpu.sync_copy(dst_hbm.at[w], didx)

        def fill(i, _):
            for jj in range(W // 16):
                onesb[i, pl.ds(jj * 16, 16)] = jnp.full((16,), 1.0, jnp.float32)
                zb[i, pl.ds(jj * 16, 16)] = jnp.zeros((16,), jnp.float32)
            return 0

        lax.fori_loop(0, K, fill, 0)
        for o, sz in cks:
            pltpu.sync_copy(zb.at[pl.ds(0, sz)], acc.at[pl.ds(base + o, sz)])
        plsc.subcore_barrier()

        def scat_start(j, b):
            pltpu.async_copy(onesb, acc.at[didx.at[j]], ssem[b], add=True)

        def scat_wait(j, b):
            pltpu.make_async_copy(onesb, acc.at[didx.at[j]], ssem[b]).wait()

        def group(g, _):
            for b in range(NSEM):
                j = g * NSEM + b

                @pl.when(j < C)
                def _():
                    @pl.when(j >= NSEM)
                    def _():
                        scat_wait(j - NSEM, b)

                    scat_start(j, b)

            return 0

        lax.fori_loop(0, -(-C // NSEM), group, 0)
        for i in range(min(NSEM, C)):
            j = C - min(NSEM, C) + i
            scat_wait(j, j % NSEM)
        plsc.subcore_barrier()
        for o, sz in cks:
            pltpu.sync_copy(
                acc.at[pl.ds(base + o, sz)],
                out_hbm.at[c].at[pl.ds(base + o, sz)],
            )

    return deg_kernel


def _make_edge_kernel(C, NP, rps, H, KE):
    cks = [(o, min(KE, rps - o)) for o in range(0, rps, KE)]

    NBUF = 2
    LOOK = 1
    NE = 2
    CE = -(-C // NE)

    @functools.partial(
        pl.kernel,
        out_type=jax.ShapeDtypeStruct((NC, NP, H), jnp.float32),
        mesh=_sc_mesh(),
        scratch_types=[
            pltpu.VMEM((CE, KE), jnp.int32),
            pltpu.VMEM((CE, KE), jnp.int32),
            pltpu.VMEM((NBUF, KE, H), jnp.float32),
            pltpu.MemorySpace.VMEM_SHARED((NP, H), jnp.float32),
            [pltpu.SemaphoreType.DMA] * NBUF,
            [pltpu.SemaphoreType.DMA] * NBUF,
        ],
    )
    def edge_kernel(zp_hbm, src_hbm, dst_hbm, out_hbm, sidx, didx, rows, acc,
                    gsem, ssem):
        c = lax.axis_index("c")
        s = lax.axis_index("s")
        w = c * NS + s
        base = s * rps

        def zfill(i, _):
            for jj in range(H // 16):
                rows[0, i, pl.ds(jj * 16, 16)] = jnp.zeros((16,), jnp.float32)
            return 0

        lax.fori_loop(0, KE, zfill, 0)
        for o, sz in cks:
            pltpu.sync_copy(rows.at[0].at[pl.ds(0, sz)],
                            acc.at[pl.ds(base + o, sz)])
        plsc.subcore_barrier()

        def gather_start(j, b):
            pltpu.async_copy(zp_hbm.at[sidx.at[j]], rows.at[b], gsem[b])

        def gather_wait(j, b):
            pltpu.make_async_copy(zp_hbm.at[sidx.at[j]], rows.at[b],
                                  gsem[b]).wait()

        def scat_start(j, b):
            pltpu.async_copy(rows.at[b], acc.at[didx.at[j]], ssem[b],
                             add=True)

        def scat_wait(j, b):
            pltpu.make_async_copy(rows.at[b], acc.at[didx.at[j]],
                                  ssem[b]).wait()

        for e in range(NE):
            ne = min(CE, C - e * CE)
            if ne <= 0:
                break
            pltpu.sync_copy(src_hbm.at[w, pl.ds(e * CE, ne)],
                            sidx.at[pl.ds(0, ne)])
            pltpu.sync_copy(dst_hbm.at[w, pl.ds(e * CE, ne)],
                            didx.at[pl.ds(0, ne)])
            for j0 in range(min(LOOK, ne)):
                gather_start(j0, j0 % NBUF)

            def group(g, _):
                for b in range(NBUF):
                    j = g * NBUF + b

                    @pl.when(j < ne)
                    def _():
                        gather_wait(j, b)
                        scat_start(j, b)
                        jn = j + LOOK
                        bn = (b + LOOK) % NBUF

                        @pl.when(jn < ne)
                        def _():
                            @pl.when(jn >= NBUF)
                            def _():
                                scat_wait(jn - NBUF, bn)

                            gather_start(jn, bn)

                return 0

            lax.fori_loop(0, -(-ne // NBUF), group, 0)
            for i in range(min(NBUF, ne)):
                j = ne - min(NBUF, ne) + i
                scat_wait(j, j % NBUF)

        plsc.subcore_barrier()
        for o, sz in cks:
            pltpu.sync_copy(
                acc.at[pl.ds(base + o, sz)],
                out_hbm.at[c].at[pl.ds(base + o, sz)],
            )

    return edge_kernel


def _dinv_block(degp):
    d = degp[0, :, 0:1] + degp[1, :, 0:1] + 1.0
    return lax.rsqrt(d)


def _tc_first(degp, x_ref, w0, zp_out):
    dinv = _dinv_block(degp)
    zp_out[...] = dinv * jnp.dot(
        x_ref[...], w0[...], preferred_element_type=jnp.float32
    )


def _tc_layer(degp, part, zp, b, gbn, beta, wn, zp_out):
    dinv = _dinv_block(degp)
    sagg = part[0] + part[1] + zp[...]
    h = jnp.maximum(dinv * sagg + b[...], 0.0)
    h = h * gbn[...] + beta[...]
    zp_out[...] = dinv * jnp.dot(h, wn[...], preferred_element_type=jnp.float32)


def _tc_final(degp, part, zp, b, gbn, beta, bat, w1, b1, w2, b2, out,
              accp, accc):
    i = pl.program_id(0)
    dinv = _dinv_block(degp)
    sagg = part[0] + part[1] + zp[...]
    h = jnp.maximum(dinv * sagg + b[...], 0.0)
    h = h * gbn[...] + beta[...]

    bt = bat[0, 0, :]
    bt2 = jnp.broadcast_to(bt[None, :], (G, BR))
    gi = lax.broadcasted_iota(jnp.int32, (G, BR), 0)
    onehot = (bt2 == gi).astype(jnp.float32)
    pp = jnp.dot(onehot, h, preferred_element_type=jnp.float32)
    cc = jnp.sum(onehot, axis=1, keepdims=True)

    @pl.when(i == 0)
    def _():
        accp[...] = pp
        accc[...] = jnp.broadcast_to(cc, accc.shape)

    @pl.when(i > 0)
    def _():
        accp[...] = accp[...] + pp
        accc[...] = accc[...] + cc

    @pl.when(i == pl.num_programs(0) - 1)
    def _():
        pooled = accp[...] / jnp.maximum(accc[...], 1.0)
        z1 = jnp.maximum(
            jnp.dot(pooled, w1[...], preferred_element_type=jnp.float32) + b1[...],
            0.0,
        )
        out[...] = (
            jnp.dot(z1, w2[...], preferred_element_type=jnp.float32) + b2[...]
        )


def kernel(x, edge_index, batch, conv_W, conv_b, bn_gamma, bn_beta,
           head_W1, head_b1, head_W2, head_b2):
    N, D = x.shape
    L, _, H = conv_W.shape
    OUT = head_W2.shape[1]
    E = edge_index.shape[1]

    KE = 128
    C = -(-E // (NW * K))
    EP = NW * K * C
    src0, dst0 = edge_index[0], edge_index[1]
    dst_r = jnp.concatenate(
        [dst0, jnp.full((EP - E,), N, jnp.int32)]).reshape(NW, C, K)
    CEdg = -(-E // (NW * KE))
    CEdg = -(-CEdg // 16) * 16
    EPe = NW * KE * CEdg
    src_r = jnp.concatenate(
        [src0, jnp.zeros((EPe - E,), jnp.int32)]).reshape(NW, CEdg, KE)
    dst_re = jnp.concatenate(
        [dst0, jnp.full((EPe - E,), N, jnp.int32)]).reshape(NW, CEdg, KE)

    rps = -(-(N + 1) // NS)
    rps = -(-rps // K) * K
    NP = NS * rps

    NB = N // BR

    deg_k = _make_deg_kernel(C, NP, rps)
    edge_k = _make_edge_kernel(CEdg, NP, rps, H, KE)

    degp = deg_k(dst_r)

    inv_s = 1.0 / math.sqrt(1.0 + EPS)
    gbn = (bn_gamma * inv_s).reshape(L, 1, H)
    beta = bn_beta.reshape(L, 1, H)
    bias = conv_b.reshape(L, 1, H)
    bat3 = batch.reshape(NB, 1, BR)

    spec_degp = pl.BlockSpec((NC, BR, 128), lambda i: (0, i, 0))
    spec_part = pl.BlockSpec((NC, BR, H), lambda i: (0, i, 0))
    spec_rows = pl.BlockSpec((BR, H), lambda i: (i, 0))
    spec_w = pl.BlockSpec((H, H), lambda i: (0, 0))
    spec_v = pl.BlockSpec((1, H), lambda i: (0, 0))

    zp = pl.pallas_call(
        _tc_first,
        grid=(NB,),
        in_specs=[spec_degp, spec_rows, spec_w],
        out_specs=spec_rows,
        out_shape=jax.ShapeDtypeStruct((N, H), jnp.float32),
    )(degp, x, conv_W[0])

    for i in range(L - 1):
        part = edge_k(zp, src_r, dst_re)
        zp = pl.pallas_call(
            _tc_layer,
            grid=(NB,),
            in_specs=[spec_degp, spec_part, spec_rows, spec_v, spec_v,
                      spec_v, spec_w],
            out_specs=spec_rows,
            out_shape=jax.ShapeDtypeStruct((N, H), jnp.float32),
        )(degp, part, zp, bias[i], gbn[i], beta[i], conv_W[i + 1])

    part = edge_k(zp, src_r, dst_re)
    out = pl.pallas_call(
        _tc_final,
        grid=(NB,),
        in_specs=[spec_degp, spec_part, spec_rows, spec_v, spec_v, spec_v,
                  pl.BlockSpec((1, 1, BR), lambda i: (i, 0, 0)),
                  spec_w, spec_v,
                  pl.BlockSpec((H, OUT), lambda i: (0, 0)),
                  pl.BlockSpec((1, OUT), lambda i: (0, 0))],
        out_specs=pl.BlockSpec((G, OUT), lambda i: (0, 0)),
        out_shape=jax.ShapeDtypeStruct((G, OUT), jnp.float32),
        scratch_shapes=[pltpu.VMEM((G, H), jnp.float32),
                        pltpu.VMEM((G, H), jnp.float32)],
    )(degp, part, zp, bias[L - 1], gbn[L - 1], beta[L - 1], bat3,
      head_W1, head_b1.reshape(1, H), head_W2, head_b2.reshape(1, OUT))
    return out


# speedup vs baseline: 3.8466x; 1.6954x over previous
import functools
import math

import jax
import jax.numpy as jnp
from jax import lax
from jax.experimental import pallas as pl
from jax.experimental.pallas import tpu as pltpu
from jax.experimental.pallas import tpu_sc as plsc

NC = 2
NS = 16
NW = NC * NS
K = 128
G = 64
EPS = 1e-5
BR = 1000


def _sc_mesh():
    return plsc.VectorSubcoreMesh(
        core_axis_name="c", subcore_axis_name="s", num_cores=NC, num_subcores=NS
    )


def _make_deg_kernel(C, NP, rps):
    W = 128
    cks = [(o, min(K, rps - o)) for o in range(0, rps, K)]

    NSEM = 4

    @functools.partial(
        pl.kernel,
        out_type=jax.ShapeDtypeStruct((NC, NP, W), jnp.float32),
        mesh=_sc_mesh(),
        scratch_types=[
            pltpu.VMEM((C, K), jnp.int32),
            pltpu.VMEM((K, W), jnp.float32),
            pltpu.VMEM((K, W), jnp.float32),
            pltpu.MemorySpace.VMEM_SHARED((NP, W), jnp.float32),
            [pltpu.SemaphoreType.DMA] * NSEM,
        ],
    )
    def deg_kernel(dst_hbm, out_hbm, didx, onesb, zb, acc, ssem):
        c = lax.axis_index("c")
        s = lax.axis_index("s")
        w = c * NS + s
        base = s * rps

        pltpu.sync_copy(dst_hbm.at[w], didx)

        def fill(i, _):
            for jj in range(W // 16):
                onesb[i, pl.ds(jj * 16, 16)] = jnp.full((16,), 1.0, jnp.float32)
                zb[i, pl.ds(jj * 16, 16)] = jnp.zeros((16,), jnp.float32)
            return 0

        lax.fori_loop(0, K, fill, 0)
        for o, sz in cks:
            pltpu.sync_copy(zb.at[pl.ds(0, sz)], acc.at[pl.ds(base + o, sz)])
        plsc.subcore_barrier()

        def scat_start(j, b):
            pltpu.async_copy(onesb, acc.at[didx.at[j]], ssem[b], add=True)

        def scat_wait(j, b):
            pltpu.make_async_copy(onesb, acc.at[didx.at[j]], ssem[b]).wait()

        def group(g, _):
            for b in range(NSEM):
                j = g * NSEM + b

                @pl.when(j < C)
                def _():
                    @pl.when(j >= NSEM)
                    def _():
                        scat_wait(j - NSEM, b)

                    scat_start(j, b)

            return 0

        lax.fori_loop(0, -(-C // NSEM), group, 0)
        for i in range(min(NSEM, C)):
            j = C - min(NSEM, C) + i
            scat_wait(j, j % NSEM)
        plsc.subcore_barrier()
        for o, sz in cks:
            pltpu.sync_copy(
                acc.at[pl.ds(base + o, sz)],
                out_hbm.at[c].at[pl.ds(base + o, sz)],
            )

    return deg_kernel


def _make_edge_kernel(C, NP, rps, H, KE):
    cks = [(o, min(KE, rps - o)) for o in range(0, rps, KE)]

    NBUF = 2
    LOOK = 1
    NE = 2
    CE = -(-C // NE)

    @functools.partial(
        pl.kernel,
        out_type=jax.ShapeDtypeStruct((NC, NP, H), jnp.float32),
        mesh=_sc_mesh(),
        scratch_types=[
            pltpu.VMEM((CE, KE), jnp.int32),
            pltpu.VMEM((CE, KE), jnp.int32),
            pltpu.VMEM((NBUF, KE, H), jnp.float32),
            pltpu.MemorySpace.VMEM_SHARED((NP, H), jnp.float32),
            [pltpu.SemaphoreType.DMA] * NBUF,
            [pltpu.SemaphoreType.DMA] * NBUF,
        ],
    )
    def edge_kernel(zp_hbm, src_hbm, dst_hbm, out_hbm, sidx, didx, rows, acc,
                    gsem, ssem):
        c = lax.axis_index("c")
        s = lax.axis_index("s")
        w = c * NS + s
        base = s * rps

        def zfill(i, _):
            for jj in range(H // 16):
                rows[0, i, pl.ds(jj * 16, 16)] = jnp.zeros((16,), jnp.float32)
            return 0

        lax.fori_loop(0, KE, zfill, 0)
        for o, sz in cks:
            pltpu.sync_copy(rows.at[0].at[pl.ds(0, sz)],
                            acc.at[pl.ds(base + o, sz)])
        plsc.subcore_barrier()

        def gather_start(j, b):
            pltpu.async_copy(zp_hbm.at[sidx.at[j]], rows.at[b], gsem[b])

        def gather_wait(j, b):
            pltpu.make_async_copy(zp_hbm.at[sidx.at[j]], rows.at[b],
                                  gsem[b]).wait()

        def scat_start(j, b):
            pltpu.async_copy(rows.at[b], acc.at[didx.at[j]], ssem[b],
                             add=True)

        def scat_wait(j, b):
            pltpu.make_async_copy(rows.at[b], acc.at[didx.at[j]],
                                  ssem[b]).wait()

        for e in range(NE):
            ne = min(CE, C - e * CE)
            if ne <= 0:
                break
            pltpu.sync_copy(src_hbm.at[w, pl.ds(e * CE, ne)],
                            sidx.at[pl.ds(0, ne)])
            pltpu.sync_copy(dst_hbm.at[w, pl.ds(e * CE, ne)],
                            didx.at[pl.ds(0, ne)])
            for j0 in range(min(LOOK, ne)):
                gather_start(j0, j0 % NBUF)

            def group(g, _):
                for b in range(NBUF):
                    j = g * NBUF + b

                    @pl.when(j < ne)
                    def _():
                        gather_wait(j, b)
                        scat_start(j, b)
                        jn = j + LOOK
                        bn = (b + LOOK) % NBUF

                        @pl.when(jn < ne)
                        def _():
                            @pl.when(jn >= NBUF)
                            def _():
                                scat_wait(jn - NBUF, bn)

                            gather_start(jn, bn)

                return 0

            lax.fori_loop(0, -(-ne // NBUF), group, 0)
            for i in range(min(NBUF, ne)):
                j = ne - min(NBUF, ne) + i
                scat_wait(j, j % NBUF)

        plsc.subcore_barrier()
        for o, sz in cks:
            pltpu.sync_copy(
                acc.at[pl.ds(base + o, sz)],
                out_hbm.at[c].at[pl.ds(base + o, sz)],
            )

    return edge_kernel


def _dinv_block(degp):
    d = degp[0, :, 0:1] + degp[1, :, 0:1] + 1.0
    return lax.rsqrt(d)


def _tc_first(degp, x_ref, w0, zp_out):
    dinv = _dinv_block(degp)
    zp_out[...] = dinv * jnp.dot(
        x_ref[...], w0[...], preferred_element_type=jnp.float32
    )


def _tc_layer(degp, part, zp, b, gbn, beta, wn, zp_out):
    dinv = _dinv_block(degp)
    sagg = part[0] + part[1] + zp[...]
    h = jnp.maximum(dinv * sagg + b[...], 0.0)
    h = h * gbn[...] + beta[...]
    zp_out[...] = dinv * jnp.dot(h, wn[...], preferred_element_type=jnp.float32)


def _tc_final(degp, part, zp, b, gbn, beta, bat, w1, b1, w2, b2, out,
              accp, accc):
    i = pl.program_id(0)
    dinv = _dinv_block(degp)
    sagg = part[0] + part[1] + zp[...]
    h = jnp.maximum(dinv * sagg + b[...], 0.0)
    h = h * gbn[...] + beta[...]

    bt = bat[0, 0, :]
    bt2 = jnp.broadcast_to(bt[None, :], (G, BR))
    gi = lax.broadcasted_iota(jnp.int32, (G, BR), 0)
    onehot = (bt2 == gi).astype(jnp.float32)
    pp = jnp.dot(onehot, h, preferred_element_type=jnp.float32)
    cc = jnp.sum(onehot, axis=1, keepdims=True)

    @pl.when(i == 0)
    def _():
        accp[...] = pp
        accc[...] = jnp.broadcast_to(cc, accc.shape)

    @pl.when(i > 0)
    def _():
        accp[...] = accp[...] + pp
        accc[...] = accc[...] + cc

    @pl.when(i == pl.num_programs(0) - 1)
    def _():
        pooled = accp[...] / jnp.maximum(accc[...], 1.0)
        z1 = jnp.maximum(
            jnp.dot(pooled, w1[...], preferred_element_type=jnp.float32) + b1[...],
            0.0,
        )
        out[...] = (
            jnp.dot(z1, w2[...], preferred_element_type=jnp.float32) + b2[...]
        )


def kernel(x, edge_index, batch, conv_W, conv_b, bn_gamma, bn_beta,
           head_W1, head_b1, head_W2, head_b2):
    N, D = x.shape
    L, _, H = conv_W.shape
    OUT = head_W2.shape[1]
    E = edge_index.shape[1]

    KE = 128
    C = -(-E // (NW * K))
    EP = NW * K * C
    src0, dst0 = edge_index[0], edge_index[1]
    dst_r = jnp.concatenate(
        [dst0, jnp.full((EP - E,), N, jnp.int32)]).reshape(NW, C, K)
    CEdg = -(-E // (NW * KE))
    EPe = NW * KE * CEdg
    src_r = jnp.concatenate(
        [src0, jnp.zeros((EPe - E,), jnp.int32)]).reshape(NW, CEdg, KE)
    dst_re = jnp.concatenate(
        [dst0, jnp.full((EPe - E,), N, jnp.int32)]).reshape(NW, CEdg, KE)

    rps = -(-(N + 1) // NS)
    rps = -(-rps // K) * K
    NP = NS * rps

    NB = N // BR

    deg_k = _make_deg_kernel(C, NP, rps)
    edge_k = _make_edge_kernel(CEdg, NP, rps, H, KE)

    degp = deg_k(dst_r)

    inv_s = 1.0 / math.sqrt(1.0 + EPS)
    gbn = (bn_gamma * inv_s).reshape(L, 1, H)
    beta = bn_beta.reshape(L, 1, H)
    bias = conv_b.reshape(L, 1, H)
    bat3 = batch.reshape(NB, 1, BR)

    spec_degp = pl.BlockSpec((NC, BR, 128), lambda i: (0, i, 0))
    spec_part = pl.BlockSpec((NC, BR, H), lambda i: (0, i, 0))
    spec_rows = pl.BlockSpec((BR, H), lambda i: (i, 0))
    spec_w = pl.BlockSpec((H, H), lambda i: (0, 0))
    spec_v = pl.BlockSpec((1, H), lambda i: (0, 0))

    zp = pl.pallas_call(
        _tc_first,
        grid=(NB,),
        in_specs=[spec_degp, spec_rows, spec_w],
        out_specs=spec_rows,
        out_shape=jax.ShapeDtypeStruct((N, H), jnp.float32),
    )(degp, x, conv_W[0])

    for i in range(L - 1):
        part = edge_k(zp, src_r, dst_re)
        zp = pl.pallas_call(
            _tc_layer,
            grid=(NB,),
            in_specs=[spec_degp, spec_part, spec_rows, spec_v, spec_v,
                      spec_v, spec_w],
            out_specs=spec_rows,
            out_shape=jax.ShapeDtypeStruct((N, H), jnp.float32),
        )(degp, part, zp, bias[i], gbn[i], beta[i], conv_W[i + 1])

    part = edge_k(zp, src_r, dst_re)
    out = pl.pallas_call(
        _tc_final,
        grid=(NB,),
        in_specs=[spec_degp, spec_part, spec_rows, spec_v, spec_v, spec_v,
                  pl.BlockSpec((1, 1, BR), lambda i: (i, 0, 0)),
                  spec_w, spec_v,
                  pl.BlockSpec((H, OUT), lambda i: (0, 0)),
                  pl.BlockSpec((1, OUT), lambda i: (0, 0))],
        out_specs=pl.BlockSpec((G, OUT), lambda i: (0, 0)),
        out_shape=jax.ShapeDtypeStruct((G, OUT), jnp.float32),
        scratch_shapes=[pltpu.VMEM((G, H), jnp.float32),
                        pltpu.VMEM((G, H), jnp.float32)],
    )(degp, part, zp, bias[L - 1], gbn[L - 1], beta[L - 1], bat3,
      head_W1, head_b1.reshape(1, H), head_W2, head_b2.reshape(1, OUT))
    return out
